# baseline (speedup 1.0000x reference)
import jax
import jax.numpy as jnp
from jax.experimental import pallas as pl
from jax.experimental.pallas import tpu as pltpu

P = 5
N = 10000
EPT = 32000
T = 6
C = 10
ED = 64
H = 64
HC = 32
DE = 4
Z = 16
L = 3
BN = 2000


def _leaky(x, s):
    return jnp.where(x >= 0, x, s * x)


def _mm_kernel(act):
    def k(h_ref, w_ref, b_ref, o_ref):
        o_ref[0] = act(jnp.dot(h_ref[0], w_ref[0],
                               preferred_element_type=jnp.float32) + b_ref[0])
    return k


def _mm3(h, W, b, act=lambda x: x):
    B, n, D = h.shape
    E = W.shape[-1]
    return pl.pallas_call(
        _mm_kernel(act),
        grid=(B, n // BN),
        in_specs=[
            pl.BlockSpec((1, BN, D), lambda i, j: (i, j, 0)),
            pl.BlockSpec((1, D, E), lambda i, j: (i, 0, 0)),
            pl.BlockSpec((1, 1, E), lambda i, j: (i, 0, 0)),
        ],
        out_specs=pl.BlockSpec((1, BN, E), lambda i, j: (i, j, 0)),
        out_shape=jax.ShapeDtypeStruct((B, n, E), jnp.float32),
    )(h, W, b[:, None, :])


def _xs_kernel(h_ref, w_ref, o_ref):
    o_ref[0] = jnp.dot(h_ref[0], w_ref[0], preferred_element_type=jnp.float32)


def _xs_all(h, gWl):
    return pl.pallas_call(
        _xs_kernel,
        grid=(P * P, N // BN),
        in_specs=[
            pl.BlockSpec((1, BN, H), lambda e, j: (e // P, j, 0)),
            pl.BlockSpec((1, H, H), lambda e, j: (e, 0, 0)),
        ],
        out_specs=pl.BlockSpec((1, BN, H), lambda e, j: (e, j, 0)),
        out_shape=jax.ShapeDtypeStruct((P * P, N, H), jnp.float32),
    )(h, gWl)


def _post_kernel(o_ref, b_ref, h_ref, xw_ref, xb_ref, cw_ref, cb_ref,
                 aw_ref, ab_ref, out_ref):
    m = jnp.max(o_ref[:, 0] + b_ref[:, 0], axis=0)
    comm = _leaky(m, 0.01)
    a = jnp.dot(h_ref[0], xw_ref[0], preferred_element_type=jnp.float32) \
        + xb_ref[0]
    b = jnp.dot(comm, cw_ref[0], preferred_element_type=jnp.float32) \
        + cb_ref[0]
    hcat = jnp.concatenate([a, b], axis=-1)
    att = jnp.dot(hcat, aw_ref[0], preferred_element_type=jnp.float32) \
        + ab_ref[0]
    out_ref[0] = hcat + jax.nn.sigmoid(att) * hcat


def _post_layer(o, gb, h, xw, xb, cw, cb, aw, ab):
    return pl.pallas_call(
        _post_kernel,
        grid=(P, N // BN),
        in_specs=[
            pl.BlockSpec((P, 1, BN, H), lambda d, j: (0, d, j, 0)),
            pl.BlockSpec((P, 1, 1, H), lambda d, j: (0, d, 0, 0)),
            pl.BlockSpec((1, BN, H), lambda d, j: (d, j, 0)),
            pl.BlockSpec((1, H, HC), lambda d, j: (d, 0, 0)),
            pl.BlockSpec((1, 1, HC), lambda d, j: (d, 0, 0)),
            pl.BlockSpec((1, H, HC), lambda d, j: (d, 0, 0)),
            pl.BlockSpec((1, 1, HC), lambda d, j: (d, 0, 0)),
            pl.BlockSpec((1, H, H), lambda d, j: (d, 0, 0)),
            pl.BlockSpec((1, 1, H), lambda d, j: (d, 0, 0)),
        ],
        out_specs=pl.BlockSpec((1, BN, H), lambda d, j: (d, j, 0)),
        out_shape=jax.ShapeDtypeStruct((P, N, H), jnp.float32),
    )(o, gb, h, xw, xb[:, None, :], cw, cb[:, None, :], aw, ab[:, None, :])


def _edge_phase(h, l, edge_index, ea_all, gat_W, gat_as, gat_ad):
    xs = _xs_all(h, gat_W[l])
    va = jnp.einsum('ehk,ek->eh', gat_W[l], gat_as[l])
    vd = jnp.einsum('ehk,ek->eh', gat_W[l], gat_ad[l])
    al_s = jnp.einsum('enh,eh->en', xs, gat_as[l])
    al_d = jnp.einsum('enh,eh->en', h[jnp.arange(P * P) % P], vd)
    src = edge_index[:, 0]
    dst = edge_index[:, 1]
    a = _leaky(jnp.take_along_axis(al_s, src, axis=1)
               + jnp.take_along_axis(al_d, dst, axis=1)
               + ea_all[l], 0.2)
    mbound = _leaky(al_s.max(axis=1) + al_d.max(axis=1)
                    + ea_all[l].max(axis=1), 0.2)
    e = jnp.exp(a - mbound[:, None])
    den = jax.vmap(lambda ee, dd: jax.ops.segment_sum(
        ee, dd, num_segments=N))(e, dst)
    w = e / (jnp.take_along_axis(den, dst, axis=1) + 1e-16)
    msgs = w[:, :, None] * jnp.take_along_axis(
        xs, src[:, :, None], axis=1)
    o = jax.vmap(lambda mm, dd: jax.ops.segment_sum(
        mm, dd, num_segments=N))(msgs, dst)
    return o.reshape(P, P, N, H)


def kernel(x, cond, edge_index, edge_attr, emb_W, emb_b, sl_W, sl_b, fin_W,
           fin_b, gat_W, gat_as, gat_ad, gat_We, gat_ae, gat_b,
           xc_W, xc_b, cc_W, cc_b, at_W, at_b):
    h = _mm3(jnp.concatenate([x, cond], axis=-1), emb_W, emb_b)
    ve = jnp.einsum('ledh,leh->led', gat_We, gat_ae)
    ea_all = jnp.einsum('ked,lkd->lke', edge_attr, ve)
    for l in range(L):
        h = _mm3(h, sl_W[l], sl_b[l], act=lambda v: _leaky(v, 0.01))
        o = _edge_phase(h, l, edge_index, ea_all, gat_W, gat_as, gat_ad)
        h = _post_layer(o, gat_b[l].reshape(P, P, 1, H), h,
                        xc_W[l], xc_b[l], cc_W[l], cc_b[l], at_W[l], at_b[l])
    out = _mm3(h, fin_W, fin_b)
    return out.reshape(P, N, Z, 2)


# speedup vs baseline: 16.0200x; 16.0200x over previous
import functools

import jax
import jax.numpy as jnp
from jax import lax
from jax.experimental import pallas as pl
from jax.experimental.pallas import tpu as pltpu
from jax.experimental.pallas import tpu_sc as plsc

P = 5
N = 10000
EPT = 32000
T = 6
C = 10
ED = 64
H = 64
HC = 32
DE = 4
Z = 16
L = 3
BN = 2000

NP = 10240
EPP = 32768
NSL = 640
NBK = 16
ETC = 13


def _leaky(x, s):
    return jnp.where(x >= 0, x, s * x)


def _mm_kernel(act):
    def k(h_ref, w_ref, b_ref, o_ref):
        o_ref[0] = act(jnp.dot(h_ref[0], w_ref[0],
                               preferred_element_type=jnp.float32) + b_ref[0])
    return k


def _mm3(h, W, b, act=lambda x: x):
    B, n, D = h.shape
    E = W.shape[-1]
    return pl.pallas_call(
        _mm_kernel(act),
        grid=(B, n // BN),
        in_specs=[
            pl.BlockSpec((1, BN, D), lambda i, j: (i, j, 0)),
            pl.BlockSpec((1, D, E), lambda i, j: (i, 0, 0)),
            pl.BlockSpec((1, 1, E), lambda i, j: (i, 0, 0)),
        ],
        out_specs=pl.BlockSpec((1, BN, E), lambda i, j: (i, j, 0)),
        out_shape=jax.ShapeDtypeStruct((B, n, E), jnp.float32),
    )(h, W, b[:, None, :])


def _xs_kernel(h_ref, w_ref, o_ref):
    o_ref[0] = jnp.dot(h_ref[0], w_ref[0], preferred_element_type=jnp.float32)


def _xs_all(h, gWl):
    return pl.pallas_call(
        _xs_kernel,
        grid=(P * P, N // BN),
        in_specs=[
            pl.BlockSpec((1, BN, H), lambda e, j: (e // P, j, 0)),
            pl.BlockSpec((1, H, H), lambda e, j: (e, 0, 0)),
        ],
        out_specs=pl.BlockSpec((1, BN, H), lambda e, j: (e, j, 0)),
        out_shape=jax.ShapeDtypeStruct((P * P, N, H), jnp.float32),
    )(h, gWl)


def _post_kernel(o_ref, b_ref, h_ref, xw_ref, xb_ref, cw_ref, cb_ref,
                 aw_ref, ab_ref, out_ref):
    m = jnp.max(o_ref[:, 0] + b_ref[:, 0], axis=0)
    comm = _leaky(m, 0.01)
    a = jnp.dot(h_ref[0], xw_ref[0], preferred_element_type=jnp.float32) \
        + xb_ref[0]
    b = jnp.dot(comm, cw_ref[0], preferred_element_type=jnp.float32) \
        + cb_ref[0]
    hcat = jnp.concatenate([a, b], axis=-1)
    att = jnp.dot(hcat, aw_ref[0], preferred_element_type=jnp.float32) \
        + ab_ref[0]
    out_ref[0] = hcat + jax.nn.sigmoid(att) * hcat


def _post_layer(o, gb, h, xw, xb, cw, cb, aw, ab):
    return pl.pallas_call(
        _post_kernel,
        grid=(P, N // BN),
        in_specs=[
            pl.BlockSpec((P, 1, BN, H), lambda d, j: (0, d, j, 0)),
            pl.BlockSpec((P, 1, 1, H), lambda d, j: (0, d, 0, 0)),
            pl.BlockSpec((1, BN, H), lambda d, j: (d, j, 0)),
            pl.BlockSpec((1, H, HC), lambda d, j: (d, 0, 0)),
            pl.BlockSpec((1, 1, HC), lambda d, j: (d, 0, 0)),
            pl.BlockSpec((1, H, HC), lambda d, j: (d, 0, 0)),
            pl.BlockSpec((1, 1, HC), lambda d, j: (d, 0, 0)),
            pl.BlockSpec((1, H, H), lambda d, j: (d, 0, 0)),
            pl.BlockSpec((1, 1, H), lambda d, j: (d, 0, 0)),
        ],
        out_specs=pl.BlockSpec((1, BN, H), lambda d, j: (d, j, 0)),
        out_shape=jax.ShapeDtypeStruct((P, N, H), jnp.float32),
    )(o, gb, h, xw, xb[:, None, :], cw, cb[:, None, :], aw, ab[:, None, :])


def _sc_edge_kernel(als, ald, msp, ea, srcg, dstl, xs, out_hbm,
                    als_v, ald_v, m_v, ea_v, srcg_v, dstl_v, e_v, den_v,
                    db_v, acc_unused, w_v, rows_v, zero_v,
                    den_parts, den_sh, out_sh, sem):
    c = lax.axis_index("c")
    s = lax.axis_index("s")
    zero16 = jnp.zeros((16,), jnp.float32)

    def _z(r, _):
        for f in range(4):
            zero_v[r, pl.ds(f * 16, 16)] = zero16
        return 0
    lax.fori_loop(0, 128, _z, 0)

    def per_et(i, _):
        et = 2 * i + c

        @pl.when(et < P * P)
        def _():
            nsl = pl.ds(s * NSL, NSL)
            pltpu.sync_copy(als.at[et], als_v)
            pltpu.sync_copy(ald.at[et], ald_v)
            pltpu.sync_copy(msp.at[et], m_v)
            pltpu.sync_copy(ea.at[et, pl.ds(s * 2048, 2048)], ea_v)
            pltpu.sync_copy(srcg.at[et, pl.ds(s * NBK, NBK)], srcg_v)
            pltpu.sync_copy(dstl.at[et, pl.ds(s * NBK, NBK)], dstl_v)

            def _zd(j, _):
                den_v[pl.ds(j * 16, 16)] = zero16
                return 0
            lax.fori_loop(0, NP // 16, _zd, 0)

            m16 = m_v[...]
            etN = et * N

            def _p1(b, _):
                for k in range(8):
                    sl = pl.ds(k * 16, 16)
                    srcv = srcg_v[b, sl] - etN
                    dstv = dstl_v[b, sl]
                    ag = plsc.load_gather(als_v, [srcv])
                    dg = plsc.load_gather(ald_v, [dstv])
                    a = ag + dg + ea_v[pl.ds(b * 128 + k * 16, 16)]
                    a = jnp.where(a >= 0.0, a, 0.2 * a)
                    e = jnp.exp(a - m16)
                    pos = lax.iota(jnp.int32, 16) + (
                        s * 2048 + b * 128 + k * 16)
                    e = jnp.where(pos < EPT, e, 0.0)
                    e_v[pl.ds(b * 128 + k * 16, 16)] = e
                    plsc.addupdate_scatter(den_v, [dstv], e)
                return 0
            lax.fori_loop(0, NBK, _p1, 0)

            pltpu.sync_copy(den_v, den_parts.at[s])
            plsc.subcore_barrier()
            for k in range(16):
                pltpu.sync_copy(den_parts.at[k, nsl], db_v.at[k])

            def _sum2(j, _):
                sl = pl.ds(j * 16, 16)
                v = db_v[0, sl]
                for k in range(1, 16):
                    v = v + db_v[k, sl]
                db_v[0, sl] = v
                return 0
            lax.fori_loop(0, NSL // 16, _sum2, 0)
            pltpu.sync_copy(db_v.at[0], den_sh.at[nsl])
            for z in range(NSL // 128):
                pltpu.sync_copy(
                    zero_v, out_sh.at[pl.ds(s * NSL + z * 128, 128)])
            plsc.subcore_barrier()
            pltpu.sync_copy(den_sh, den_v)

            def _p2(b, _):
                pltpu.async_copy(xs.at[srcg_v.at[b]], rows_v, sem).wait()
                for k in range(8):
                    sl = pl.ds(k * 16, 16)
                    dstv = dstl_v[b, sl]
                    dg = plsc.load_gather(den_v, [dstv])
                    w_v[sl] = e_v[pl.ds(b * 128 + k * 16, 16)] / (
                        dg + 1e-16)

                def _scale(i2, _):
                    wsp = plsc.load_gather(
                        w_v, [jnp.full((16,), i2, jnp.int32)])
                    for f in range(4):
                        fsl = pl.ds(f * 16, 16)
                        rows_v[i2, fsl] = rows_v[i2, fsl] * wsp
                    return 0
                lax.fori_loop(0, 128, _scale, 0)
                pltpu.sync_copy(rows_v, out_sh.at[dstl_v.at[b]], add=True)
                return 0
            lax.fori_loop(0, NBK, _p2, 0)
            plsc.subcore_barrier()
            for z in range(NSL // 128):
                r0 = s * NSL + z * 128
                pltpu.sync_copy(out_sh.at[pl.ds(r0, 128)],
                                out_hbm.at[et, pl.ds(r0, 128)])
            plsc.subcore_barrier()
        return 0

    lax.fori_loop(0, ETC, per_et, 0)


def _sc_edge_call(als, ald, msp, ea, srcg, dstl, xs_flat):
    f32 = jnp.float32
    mesh = plsc.VectorSubcoreMesh(core_axis_name="c", subcore_axis_name="s")
    return pl.kernel(
        _sc_edge_kernel,
        out_type=jax.ShapeDtypeStruct((P * P, NP, H), f32),
        mesh=mesh,
        compiler_params=pltpu.CompilerParams(needs_layout_passes=False,
                                             use_tc_tiling_on_sc=False),
        scratch_types=[
            pltpu.VMEM((NP,), f32),
            pltpu.VMEM((NP,), f32),
            pltpu.VMEM((16,), f32),
            pltpu.VMEM((2048,), f32),
            pltpu.VMEM((NBK, 128), jnp.int32),
            pltpu.VMEM((NBK, 128), jnp.int32),
            pltpu.VMEM((2048,), f32),
            pltpu.VMEM((NP,), f32),
            pltpu.VMEM((16, NSL), f32),
            pltpu.VMEM((16,), f32),
            pltpu.VMEM((128,), f32),
            pltpu.VMEM((128, H), f32),
            pltpu.VMEM((128, H), f32),
            pltpu.VMEM_SHARED((16, NP), f32),
            pltpu.VMEM_SHARED((NP,), f32),
            pltpu.VMEM_SHARED((NP, H), f32),
            pltpu.SemaphoreType.DMA,
        ],
    )(als, ald, msp, ea, srcg, dstl, xs_flat)


def _edge_phase(h, l, ea_pad, srcg, dstl, gat_W, gat_as, gat_ad):
    xs = _xs_all(h, gat_W[l])
    vd = jnp.einsum('ehk,ek->eh', gat_W[l], gat_ad[l])
    al_s = jnp.einsum('enh,eh->en', xs, gat_as[l])
    al_d = jnp.einsum('enh,eh->en', h[jnp.arange(P * P) % P], vd)
    m = _leaky(al_s.max(1) + al_d.max(1) + ea_pad[l].max(1), 0.2)
    msp = jnp.broadcast_to(m[:, None], (P * P, 16))
    als_p = jnp.pad(al_s, ((0, 0), (0, NP - N)))
    ald_p = jnp.pad(al_d, ((0, 0), (0, NP - N)))
    out = _sc_edge_call(als_p, ald_p, msp, ea_pad[l], srcg, dstl,
                        xs.reshape(P * P * N, H))
    return out[:, :N, :].reshape(P, P, N, H)


def kernel(x, cond, edge_index, edge_attr, emb_W, emb_b, sl_W, sl_b, fin_W,
           fin_b, gat_W, gat_as, gat_ad, gat_We, gat_ae, gat_b,
           xc_W, xc_b, cc_W, cc_b, at_W, at_b):
    h = _mm3(jnp.concatenate([x, cond], axis=-1), emb_W, emb_b)
    ve = jnp.einsum('ledh,leh->led', gat_We, gat_ae)
    ea_all = jnp.einsum('ked,lkd->lke', edge_attr, ve)
    ea_pad = jnp.pad(ea_all, ((0, 0), (0, 0), (0, EPP - EPT)))
    et_off = (jnp.arange(P * P, dtype=jnp.int32) * N)[:, None]
    srcg = (jnp.pad(edge_index[:, 0], ((0, 0), (0, EPP - EPT)))
            + et_off).reshape(P * P, EPP // 128, 128)
    dstl = jnp.pad(edge_index[:, 1],
                   ((0, 0), (0, EPP - EPT))).reshape(P * P, EPP // 128, 128)
    for l in range(L):
        h = _mm3(h, sl_W[l], sl_b[l], act=lambda v: _leaky(v, 0.01))
        o = _edge_phase(h, l, ea_pad, srcg, dstl, gat_W, gat_as, gat_ad)
        h = _post_layer(o, gat_b[l].reshape(P, P, 1, H), h,
                        xc_W[l], xc_b[l], cc_W[l], cc_b[l], at_W[l], at_b[l])
    out = _mm3(h, fin_W, fin_b)
    return out.reshape(P, N, Z, 2)


# speedup vs baseline: 18.7333x; 1.1694x over previous
import functools

import jax
import jax.numpy as jnp
from jax import lax
from jax.experimental import pallas as pl
from jax.experimental.pallas import tpu as pltpu
from jax.experimental.pallas import tpu_sc as plsc

P = 5
N = 10000
EPT = 32000
T = 6
C = 10
ED = 64
H = 64
HC = 32
DE = 4
Z = 16
L = 3
BN = 2000

NP = 10240
EPP = 32768
NSL = 640
NBK = 16
ETC = 13


def _leaky(x, s):
    return jnp.where(x >= 0, x, s * x)


def _mm_kernel(act):
    def k(h_ref, w_ref, b_ref, o_ref):
        o_ref[0] = act(jnp.dot(h_ref[0], w_ref[0],
                               preferred_element_type=jnp.float32) + b_ref[0])
    return k


def _mm3(h, W, b, act=lambda x: x):
    B, n, D = h.shape
    E = W.shape[-1]
    return pl.pallas_call(
        _mm_kernel(act),
        grid=(B, n // BN),
        in_specs=[
            pl.BlockSpec((1, BN, D), lambda i, j: (i, j, 0)),
            pl.BlockSpec((1, D, E), lambda i, j: (i, 0, 0)),
            pl.BlockSpec((1, 1, E), lambda i, j: (i, 0, 0)),
        ],
        out_specs=pl.BlockSpec((1, BN, E), lambda i, j: (i, j, 0)),
        out_shape=jax.ShapeDtypeStruct((B, n, E), jnp.float32),
    )(h, W, b[:, None, :])


def _xs_kernel(h_ref, w_ref, o_ref):
    o_ref[0] = jnp.dot(h_ref[0], w_ref[0], preferred_element_type=jnp.float32)


def _xs_all(h, gWl):
    return pl.pallas_call(
        _xs_kernel,
        grid=(P * P, N // BN),
        in_specs=[
            pl.BlockSpec((1, BN, H), lambda e, j: (e // P, j, 0)),
            pl.BlockSpec((1, H, H), lambda e, j: (e, 0, 0)),
        ],
        out_specs=pl.BlockSpec((1, BN, H), lambda e, j: (e, j, 0)),
        out_shape=jax.ShapeDtypeStruct((P * P, N, H), jnp.float32),
    )(h, gWl)


def _post_kernel(o_ref, b_ref, h_ref, xw_ref, xb_ref, cw_ref, cb_ref,
                 aw_ref, ab_ref, out_ref):
    m = jnp.max(o_ref[:, 0] + b_ref[:, 0], axis=0)
    comm = _leaky(m, 0.01)
    a = jnp.dot(h_ref[0], xw_ref[0], preferred_element_type=jnp.float32) \
        + xb_ref[0]
    b = jnp.dot(comm, cw_ref[0], preferred_element_type=jnp.float32) \
        + cb_ref[0]
    hcat = jnp.concatenate([a, b], axis=-1)
    att = jnp.dot(hcat, aw_ref[0], preferred_element_type=jnp.float32) \
        + ab_ref[0]
    out_ref[0] = hcat + jax.nn.sigmoid(att) * hcat


def _post_layer(o, gb, h, xw, xb, cw, cb, aw, ab):
    return pl.pallas_call(
        _post_kernel,
        grid=(P, N // BN),
        in_specs=[
            pl.BlockSpec((P, 1, BN, H), lambda d, j: (0, d, j, 0)),
            pl.BlockSpec((P, 1, 1, H), lambda d, j: (0, d, 0, 0)),
            pl.BlockSpec((1, BN, H), lambda d, j: (d, j, 0)),
            pl.BlockSpec((1, H, HC), lambda d, j: (d, 0, 0)),
            pl.BlockSpec((1, 1, HC), lambda d, j: (d, 0, 0)),
            pl.BlockSpec((1, H, HC), lambda d, j: (d, 0, 0)),
            pl.BlockSpec((1, 1, HC), lambda d, j: (d, 0, 0)),
            pl.BlockSpec((1, H, H), lambda d, j: (d, 0, 0)),
            pl.BlockSpec((1, 1, H), lambda d, j: (d, 0, 0)),
        ],
        out_specs=pl.BlockSpec((1, BN, H), lambda d, j: (d, j, 0)),
        out_shape=jax.ShapeDtypeStruct((P, N, H), jnp.float32),
    )(o, gb, h, xw, xb[:, None, :], cw, cb[:, None, :], aw, ab[:, None, :])


def _sc_edge_kernel(als, ald, msp, ea, srcg, dstl, xs, out_hbm,
                    als_v, ald_v, m_v, ea_v, srcg_v, dstl_v, e_v, den_v,
                    db_v, w_v, rows_v, rows2_v, zero_v,
                    den_parts, den_sh, out_sh, sem, sem2):
    c = lax.axis_index("c")
    s = lax.axis_index("s")
    zero16 = jnp.zeros((16,), jnp.float32)

    def _z(r, _):
        for f in range(4):
            zero_v[r, pl.ds(f * 16, 16)] = zero16
        return 0
    lax.fori_loop(0, 128, _z, 0)

    def per_et(i, _):
        et = 2 * i + c

        @pl.when(et < P * P)
        def _():
            nsl = pl.ds(s * NSL, NSL)
            pltpu.sync_copy(als.at[et], als_v)
            pltpu.sync_copy(ald.at[et], ald_v)
            pltpu.sync_copy(msp.at[et], m_v)
            pltpu.sync_copy(ea.at[et, pl.ds(s * 2048, 2048)], ea_v)
            pltpu.sync_copy(srcg.at[et, pl.ds(s * NBK, NBK)], srcg_v)
            pltpu.sync_copy(dstl.at[et, pl.ds(s * NBK, NBK)], dstl_v)

            def _zd(j, _):
                den_v[pl.ds(j * 16, 16)] = zero16
                return 0
            lax.fori_loop(0, NP // 16, _zd, 0)

            m16 = m_v[...]
            etN = et * N

            def _p1(b, _):
                for k in range(8):
                    sl = pl.ds(k * 16, 16)
                    srcv = srcg_v[b, sl] - etN
                    dstv = dstl_v[b, sl]
                    ag = plsc.load_gather(als_v, [srcv])
                    dg = plsc.load_gather(ald_v, [dstv])
                    a = ag + dg + ea_v[pl.ds(b * 128 + k * 16, 16)]
                    a = jnp.where(a >= 0.0, a, 0.2 * a)
                    e = jnp.exp(a - m16)
                    pos = lax.iota(jnp.int32, 16) + (
                        s * 2048 + b * 128 + k * 16)
                    e = jnp.where(pos < EPT, e, 0.0)
                    e_v[pl.ds(b * 128 + k * 16, 16)] = e
                    plsc.addupdate_scatter(den_v, [dstv], e)
                return 0
            lax.fori_loop(0, NBK, _p1, 0)

            pltpu.sync_copy(den_v, den_parts.at[s])
            plsc.subcore_barrier()
            for k in range(16):
                pltpu.sync_copy(den_parts.at[k, nsl], db_v.at[k])

            def _sum2(j, _):
                sl = pl.ds(j * 16, 16)
                v = db_v[0, sl]
                for k in range(1, 16):
                    v = v + db_v[k, sl]
                db_v[0, sl] = v
                return 0
            lax.fori_loop(0, NSL // 16, _sum2, 0)
            pltpu.sync_copy(db_v.at[0], den_sh.at[nsl])
            for z in range(NSL // 128):
                pltpu.sync_copy(
                    zero_v, out_sh.at[pl.ds(s * NSL + z * 128, 128)])
            plsc.subcore_barrier()
            pltpu.sync_copy(den_sh, den_v)

            def _w_and_scale(b, buf):
                for k in range(8):
                    sl = pl.ds(k * 16, 16)
                    dstv = dstl_v[b, sl]
                    dg = plsc.load_gather(den_v, [dstv])
                    w_v[sl] = e_v[pl.ds(b * 128 + k * 16, 16)] / (
                        dg + 1e-16)

                def _scale(i2, _):
                    wsp = plsc.load_gather(
                        w_v, [jnp.full((16,), i2, jnp.int32)])
                    for f in range(4):
                        fsl = pl.ds(f * 16, 16)
                        buf[i2, fsl] = buf[i2, fsl] * wsp
                    return 0
                lax.fori_loop(0, 128, _scale, 0)

            bufs = (rows_v, rows2_v)
            sems = (sem, sem2)
            handles = [pltpu.async_copy(xs.at[srcg_v.at[0]], bufs[0],
                                        sems[0]), None]
            for b in range(NBK):
                if b + 1 < NBK:
                    handles[(b + 1) % 2] = pltpu.async_copy(
                        xs.at[srcg_v.at[b + 1]], bufs[(b + 1) % 2],
                        sems[(b + 1) % 2])
                handles[b % 2].wait()
                _w_and_scale(b, bufs[b % 2])
                pltpu.sync_copy(bufs[b % 2], out_sh.at[dstl_v.at[b]],
                                add=True)
            plsc.subcore_barrier()
            for z in range(NSL // 128):
                r0 = s * NSL + z * 128
                pltpu.sync_copy(out_sh.at[pl.ds(r0, 128)],
                                out_hbm.at[et, pl.ds(r0, 128)])
            plsc.subcore_barrier()
        return 0

    lax.fori_loop(0, ETC, per_et, 0)


def _sc_edge_call(als, ald, msp, ea, srcg, dstl, xs_flat):
    f32 = jnp.float32
    mesh = plsc.VectorSubcoreMesh(core_axis_name="c", subcore_axis_name="s")
    return pl.kernel(
        _sc_edge_kernel,
        out_type=jax.ShapeDtypeStruct((P * P, NP, H), f32),
        mesh=mesh,
        compiler_params=pltpu.CompilerParams(needs_layout_passes=False,
                                             use_tc_tiling_on_sc=False),
        scratch_types=[
            pltpu.VMEM((NP,), f32),
            pltpu.VMEM((NP,), f32),
            pltpu.VMEM((16,), f32),
            pltpu.VMEM((2048,), f32),
            pltpu.VMEM((NBK, 128), jnp.int32),
            pltpu.VMEM((NBK, 128), jnp.int32),
            pltpu.VMEM((2048,), f32),
            pltpu.VMEM((NP,), f32),
            pltpu.VMEM((16, NSL), f32),
            pltpu.VMEM((128,), f32),
            pltpu.VMEM((128, H), f32),
            pltpu.VMEM((128, H), f32),
            pltpu.VMEM((128, H), f32),
            pltpu.VMEM_SHARED((16, NP), f32),
            pltpu.VMEM_SHARED((NP,), f32),
            pltpu.VMEM_SHARED((NP, H), f32),
            pltpu.SemaphoreType.DMA,
            pltpu.SemaphoreType.DMA,
        ],
    )(als, ald, msp, ea, srcg, dstl, xs_flat)


def _edge_phase(h, l, ea_pad, srcg, dstl, gat_W, gat_as, gat_ad):
    xs = _xs_all(h, gat_W[l])
    vd = jnp.einsum('ehk,ek->eh', gat_W[l], gat_ad[l])
    al_s = jnp.einsum('enh,eh->en', xs, gat_as[l])
    al_d = jnp.einsum('enh,eh->en', h[jnp.arange(P * P) % P], vd)
    m = _leaky(al_s.max(1) + al_d.max(1) + ea_pad[l].max(1), 0.2)
    msp = jnp.broadcast_to(m[:, None], (P * P, 16))
    als_p = jnp.pad(al_s, ((0, 0), (0, NP - N)))
    ald_p = jnp.pad(al_d, ((0, 0), (0, NP - N)))
    out = _sc_edge_call(als_p, ald_p, msp, ea_pad[l], srcg, dstl,
                        xs.reshape(P * P * N, H))
    return out[:, :N, :].reshape(P, P, N, H)


def kernel(x, cond, edge_index, edge_attr, emb_W, emb_b, sl_W, sl_b, fin_W,
           fin_b, gat_W, gat_as, gat_ad, gat_We, gat_ae, gat_b,
           xc_W, xc_b, cc_W, cc_b, at_W, at_b):
    h = _mm3(jnp.concatenate([x, cond], axis=-1), emb_W, emb_b)
    ve = jnp.einsum('ledh,leh->led', gat_We, gat_ae)
    ea_all = jnp.einsum('ked,lkd->lke', edge_attr, ve)
    ea_pad = jnp.pad(ea_all, ((0, 0), (0, 0), (0, EPP - EPT)))
    et_off = (jnp.arange(P * P, dtype=jnp.int32) * N)[:, None]
    srcg = (jnp.pad(edge_index[:, 0], ((0, 0), (0, EPP - EPT)))
            + et_off).reshape(P * P, EPP // 128, 128)
    dstl = jnp.pad(edge_index[:, 1],
                   ((0, 0), (0, EPP - EPT))).reshape(P * P, EPP // 128, 128)
    for l in range(L):
        h = _mm3(h, sl_W[l], sl_b[l], act=lambda v: _leaky(v, 0.01))
        o = _edge_phase(h, l, ea_pad, srcg, dstl, gat_W, gat_as, gat_ad)
        h = _post_layer(o, gat_b[l].reshape(P, P, 1, H), h,
                        xc_W[l], xc_b[l], cc_W[l], cc_b[l], at_W[l], at_b[l])
    out = _mm3(h, fin_W, fin_b)
    return out.reshape(P, N, Z, 2)


# speedup vs baseline: 19.6311x; 1.0479x over previous
import functools

import jax
import jax.numpy as jnp
from jax import lax
from jax.experimental import pallas as pl
from jax.experimental.pallas import tpu as pltpu
from jax.experimental.pallas import tpu_sc as plsc

P = 5
N = 10000
EPT = 32000
T = 6
C = 10
ED = 64
H = 64
HC = 32
DE = 4
Z = 16
L = 3
BN = 2000

NP = 10240
EPP = 32768
NSL = 640
NBK = 16
ETC = 13


def _leaky(x, s):
    return jnp.where(x >= 0, x, s * x)


def _mm_kernel(act):
    def k(h_ref, w_ref, b_ref, o_ref):
        o_ref[0] = act(jnp.dot(h_ref[0], w_ref[0],
                               preferred_element_type=jnp.float32) + b_ref[0])
    return k


def _mm3(h, W, b, act=lambda x: x):
    B, n, D = h.shape
    E = W.shape[-1]
    return pl.pallas_call(
        _mm_kernel(act),
        grid=(B, n // BN),
        in_specs=[
            pl.BlockSpec((1, BN, D), lambda i, j: (i, j, 0)),
            pl.BlockSpec((1, D, E), lambda i, j: (i, 0, 0)),
            pl.BlockSpec((1, 1, E), lambda i, j: (i, 0, 0)),
        ],
        out_specs=pl.BlockSpec((1, BN, E), lambda i, j: (i, j, 0)),
        out_shape=jax.ShapeDtypeStruct((B, n, E), jnp.float32),
    )(h, W, b[:, None, :])


def _xs_kernel(h_ref, w_ref, o_ref):
    o_ref[0] = jnp.dot(h_ref[0], w_ref[0], preferred_element_type=jnp.float32)


def _xs_all(h, gWl):
    return pl.pallas_call(
        _xs_kernel,
        grid=(P * P, N // BN),
        in_specs=[
            pl.BlockSpec((1, BN, H), lambda e, j: (e // P, j, 0)),
            pl.BlockSpec((1, H, H), lambda e, j: (e, 0, 0)),
        ],
        out_specs=pl.BlockSpec((1, BN, H), lambda e, j: (e, j, 0)),
        out_shape=jax.ShapeDtypeStruct((P * P, N, H), jnp.float32),
    )(h, gWl)


def _post_kernel(o_ref, b_ref, h_ref, xw_ref, xb_ref, cw_ref, cb_ref,
                 aw_ref, ab_ref, out_ref):
    m = jnp.max(o_ref[:, 0] + b_ref[:, 0], axis=0)
    comm = _leaky(m, 0.01)
    a = jnp.dot(h_ref[0], xw_ref[0], preferred_element_type=jnp.float32) \
        + xb_ref[0]
    b = jnp.dot(comm, cw_ref[0], preferred_element_type=jnp.float32) \
        + cb_ref[0]
    hcat = jnp.concatenate([a, b], axis=-1)
    att = jnp.dot(hcat, aw_ref[0], preferred_element_type=jnp.float32) \
        + ab_ref[0]
    out_ref[0] = hcat + jax.nn.sigmoid(att) * hcat


def _post_layer(o, gb, h, xw, xb, cw, cb, aw, ab):
    return pl.pallas_call(
        _post_kernel,
        grid=(P, N // BN),
        in_specs=[
            pl.BlockSpec((P, 1, BN, H), lambda d, j: (0, d, j, 0)),
            pl.BlockSpec((P, 1, 1, H), lambda d, j: (0, d, 0, 0)),
            pl.BlockSpec((1, BN, H), lambda d, j: (d, j, 0)),
            pl.BlockSpec((1, H, HC), lambda d, j: (d, 0, 0)),
            pl.BlockSpec((1, 1, HC), lambda d, j: (d, 0, 0)),
            pl.BlockSpec((1, H, HC), lambda d, j: (d, 0, 0)),
            pl.BlockSpec((1, 1, HC), lambda d, j: (d, 0, 0)),
            pl.BlockSpec((1, H, H), lambda d, j: (d, 0, 0)),
            pl.BlockSpec((1, 1, H), lambda d, j: (d, 0, 0)),
        ],
        out_specs=pl.BlockSpec((1, BN, H), lambda d, j: (d, j, 0)),
        out_shape=jax.ShapeDtypeStruct((P, N, H), jnp.float32),
    )(o, gb, h, xw, xb[:, None, :], cw, cb[:, None, :], aw, ab[:, None, :])


def _sc_edge_kernel(als, ald, msp, ea, srcg, dstl, xs, out_hbm,
                    als_v, ald_v, m_v, ea_v, srcg_v, dstl_v, e_v, den_v,
                    iota_v, zden_v, w_v, rows_v, zero_v,
                    den_sh, out_sh,
                    g0, g1, g2, g3, s0, s1, s2, s3):
    gsems = (g0, g1, g2, g3)
    ssems = (s0, s1, s2, s3)
    c = lax.axis_index("c")
    s = lax.axis_index("s")
    zero16 = jnp.zeros((16,), jnp.float32)
    iota16 = lax.iota(jnp.int32, 16)

    def _z(r, _):
        for f in range(4):
            zero_v[r, pl.ds(f * 16, 16)] = zero16
        return 0
    lax.fori_loop(0, 128, _z, 0)

    def _z2(r, _):
        zden_v[r, :] = zero16
        return 0
    lax.fori_loop(0, 40, _z2, 0)
    for z in range(5):
        for j in range(8):
            iota_v[z, pl.ds(j * 16, 16)] = iota16 + (z * 128 + j * 16)

    def per_et(i, _):
        et = 2 * i + c

        @pl.when(et < P * P)
        def _():
            nsl = pl.ds(s * NSL, NSL)
            pltpu.sync_copy(als.at[et], als_v)
            pltpu.sync_copy(ald.at[et], ald_v)
            pltpu.sync_copy(msp.at[et], m_v)
            pltpu.sync_copy(ea.at[et, pl.ds(s * 2048, 2048)], ea_v)
            pltpu.sync_copy(srcg.at[et, pl.ds(s * NBK, NBK)], srcg_v)
            pltpu.sync_copy(dstl.at[et, pl.ds(s * NBK, NBK)], dstl_v)

            def _zd(r, _):
                den_v[r, :] = zero16
                return 0
            lax.fori_loop(0, NP // 16, _zd, 0)
            pltpu.sync_copy(zden_v, den_sh.at[pl.ds(s * 40, 40)])
            for z in range(NSL // 128):
                pltpu.sync_copy(
                    zero_v, out_sh.at[pl.ds(s * NSL + z * 128, 128)])
            plsc.subcore_barrier()

            m16 = m_v[...]
            etN = et * N

            def _p1(b, _):
                for k in range(8):
                    sl = pl.ds(k * 16, 16)
                    srcv = srcg_v[b, sl] - etN
                    dstv = dstl_v[b, sl]
                    ag = plsc.load_gather(als_v, [srcv])
                    dg = plsc.load_gather(ald_v, [dstv])
                    a = ag + dg + ea_v[pl.ds(b * 128 + k * 16, 16)]
                    a = jnp.where(a >= 0.0, a, 0.2 * a)
                    e = jnp.exp(a - m16)
                    pos = lax.iota(jnp.int32, 16) + (
                        s * 2048 + b * 128 + k * 16)
                    e = jnp.where(pos < EPT, e, 0.0)
                    e_v[pl.ds(b * 128 + k * 16, 16)] = e
                    plsc.addupdate_scatter(
                        den_v, [dstv >> 4, dstv & 15], e)
                return 0
            lax.fori_loop(0, NBK, _p1, 0)

            for z in range(5):
                pltpu.sync_copy(den_v.at[pl.ds(z * 128, 128)],
                                den_sh.at[iota_v.at[z]], add=True)
            plsc.subcore_barrier()
            pltpu.sync_copy(den_sh, den_v)

            def _w_and_scale(b, buf):
                for k in range(8):
                    sl = pl.ds(k * 16, 16)
                    dstv = dstl_v[b, sl]
                    dg = plsc.load_gather(den_v, [dstv >> 4, dstv & 15])
                    w_v[sl] = e_v[pl.ds(b * 128 + k * 16, 16)] / (
                        dg + 1e-16)

                def _scale(i2, _):
                    wsp = plsc.load_gather(
                        w_v, [jnp.full((16,), i2, jnp.int32)])
                    for f in range(4):
                        fsl = pl.ds(f * 16, 16)
                        buf[i2, fsl] = buf[i2, fsl] * wsp
                    return 0
                lax.fori_loop(0, 128, _scale, 0)

            gh = [None] * 4
            sh = [None] * 4
            for b in (0, 1):
                gh[b] = pltpu.async_copy(xs.at[srcg_v.at[b]], rows_v.at[b],
                                         gsems[b])
            for b in range(NBK):
                i = b % 4
                gh[i].wait()
                _w_and_scale(b, rows_v.at[i])
                sh[i] = pltpu.async_copy(rows_v.at[i],
                                         out_sh.at[dstl_v.at[b]],
                                         ssems[i], add=True)
                if b + 2 < NBK:
                    j = (b + 2) % 4
                    if sh[j] is not None:
                        sh[j].wait()
                    gh[j] = pltpu.async_copy(xs.at[srcg_v.at[b + 2]],
                                             rows_v.at[j], gsems[j])
            for i in range(4):
                sh[i].wait()
            plsc.subcore_barrier()
            for z in range(NSL // 128):
                r0 = s * NSL + z * 128
                pltpu.sync_copy(out_sh.at[pl.ds(r0, 128)],
                                out_hbm.at[et, pl.ds(r0, 128)])
            plsc.subcore_barrier()
        return 0

    lax.fori_loop(0, ETC, per_et, 0)


def _sc_edge_call(als, ald, msp, ea, srcg, dstl, xs_flat):
    f32 = jnp.float32
    mesh = plsc.VectorSubcoreMesh(core_axis_name="c", subcore_axis_name="s")
    return pl.kernel(
        _sc_edge_kernel,
        out_type=jax.ShapeDtypeStruct((P * P, NP, H), f32),
        mesh=mesh,
        compiler_params=pltpu.CompilerParams(needs_layout_passes=False,
                                             use_tc_tiling_on_sc=False),
        scratch_types=[
            pltpu.VMEM((NP,), f32),
            pltpu.VMEM((NP,), f32),
            pltpu.VMEM((16,), f32),
            pltpu.VMEM((2048,), f32),
            pltpu.VMEM((NBK, 128), jnp.int32),
            pltpu.VMEM((NBK, 128), jnp.int32),
            pltpu.VMEM((2048,), f32),
            pltpu.VMEM((NP // 16, 16), f32),
            pltpu.VMEM((5, 128), jnp.int32),
            pltpu.VMEM((40, 16), f32),
            pltpu.VMEM((128,), f32),
            pltpu.VMEM((4, 128, H), f32),
            pltpu.VMEM((128, H), f32),
            pltpu.VMEM_SHARED((NP // 16, 16), f32),
            pltpu.VMEM_SHARED((NP, H), f32),
        ] + [pltpu.SemaphoreType.DMA] * 8,
    )(als, ald, msp, ea, srcg, dstl, xs_flat)


def _edge_phase(h, l, ea_pad, srcg, dstl, gat_W, gat_as, gat_ad):
    xs = _xs_all(h, gat_W[l])
    vd = jnp.einsum('ehk,ek->eh', gat_W[l], gat_ad[l])
    al_s = jnp.einsum('enh,eh->en', xs, gat_as[l])
    al_d = jnp.einsum('enh,eh->en', h[jnp.arange(P * P) % P], vd)
    m = _leaky(al_s.max(1) + al_d.max(1) + ea_pad[l].max(1), 0.2)
    msp = jnp.broadcast_to(m[:, None], (P * P, 16))
    als_p = jnp.pad(al_s, ((0, 0), (0, NP - N)))
    ald_p = jnp.pad(al_d, ((0, 0), (0, NP - N)))
    out = _sc_edge_call(als_p, ald_p, msp, ea_pad[l], srcg, dstl,
                        xs.reshape(P * P * N, H))
    return out[:, :N, :].reshape(P, P, N, H)


def kernel(x, cond, edge_index, edge_attr, emb_W, emb_b, sl_W, sl_b, fin_W,
           fin_b, gat_W, gat_as, gat_ad, gat_We, gat_ae, gat_b,
           xc_W, xc_b, cc_W, cc_b, at_W, at_b):
    h = _mm3(jnp.concatenate([x, cond], axis=-1), emb_W, emb_b)
    ve = jnp.einsum('ledh,leh->led', gat_We, gat_ae)
    ea_all = jnp.einsum('ked,lkd->lke', edge_attr, ve)
    ea_pad = jnp.pad(ea_all, ((0, 0), (0, 0), (0, EPP - EPT)))
    et_off = (jnp.arange(P * P, dtype=jnp.int32) * N)[:, None]
    srcg = (jnp.pad(edge_index[:, 0], ((0, 0), (0, EPP - EPT)))
            + et_off).reshape(P * P, EPP // 128, 128)
    dstl = jnp.pad(edge_index[:, 1],
                   ((0, 0), (0, EPP - EPT))).reshape(P * P, EPP // 128, 128)
    for l in range(L):
        h = _mm3(h, sl_W[l], sl_b[l], act=lambda v: _leaky(v, 0.01))
        o = _edge_phase(h, l, ea_pad, srcg, dstl, gat_W, gat_as, gat_ad)
        h = _post_layer(o, gat_b[l].reshape(P, P, 1, H), h,
                        xc_W[l], xc_b[l], cc_W[l], cc_b[l], at_W[l], at_b[l])
    out = _mm3(h, fin_W, fin_b)
    return out.reshape(P, N, Z, 2)


# speedup vs baseline: 21.3273x; 1.0864x over previous
import functools

import jax
import jax.numpy as jnp
from jax import lax
from jax.experimental import pallas as pl
from jax.experimental.pallas import tpu as pltpu
from jax.experimental.pallas import tpu_sc as plsc

P = 5
N = 10000
EPT = 32000
T = 6
C = 10
ED = 64
H = 64
HC = 32
DE = 4
Z = 16
L = 3
BN = 2000

NP = 10240
EPP = 32768
NSL = 640
NBK = 16
ETC = 13


def _leaky(x, s):
    return jnp.where(x >= 0, x, s * x)


def _mm_kernel(act):
    def k(h_ref, w_ref, b_ref, o_ref):
        o_ref[0] = act(jnp.dot(h_ref[0], w_ref[0],
                               preferred_element_type=jnp.float32) + b_ref[0])
    return k


def _mm3(h, W, b, act=lambda x: x):
    B, n, D = h.shape
    E = W.shape[-1]
    return pl.pallas_call(
        _mm_kernel(act),
        grid=(B, n // BN),
        in_specs=[
            pl.BlockSpec((1, BN, D), lambda i, j: (i, j, 0)),
            pl.BlockSpec((1, D, E), lambda i, j: (i, 0, 0)),
            pl.BlockSpec((1, 1, E), lambda i, j: (i, 0, 0)),
        ],
        out_specs=pl.BlockSpec((1, BN, E), lambda i, j: (i, j, 0)),
        out_shape=jax.ShapeDtypeStruct((B, n, E), jnp.float32),
    )(h, W, b[:, None, :])


def _xs_kernel(h_ref, w_ref, o_ref):
    o_ref[0] = jnp.dot(h_ref[0], w_ref[0], preferred_element_type=jnp.float32)


def _xs_all(h, gWl):
    return pl.pallas_call(
        _xs_kernel,
        grid=(P * P, N // BN),
        in_specs=[
            pl.BlockSpec((1, BN, H), lambda e, j: (e // P, j, 0)),
            pl.BlockSpec((1, H, H), lambda e, j: (e, 0, 0)),
        ],
        out_specs=pl.BlockSpec((1, BN, H), lambda e, j: (e, j, 0)),
        out_shape=jax.ShapeDtypeStruct((P * P, N, H), jnp.float32),
    )(h, gWl)


def _post_kernel(o_ref, b_ref, h_ref, xw_ref, xb_ref, cw_ref, cb_ref,
                 aw_ref, ab_ref, out_ref):
    m = jnp.max(o_ref[:, 0] + b_ref[:, 0], axis=0)
    comm = _leaky(m, 0.01)
    a = jnp.dot(h_ref[0], xw_ref[0], preferred_element_type=jnp.float32) \
        + xb_ref[0]
    b = jnp.dot(comm, cw_ref[0], preferred_element_type=jnp.float32) \
        + cb_ref[0]
    hcat = jnp.concatenate([a, b], axis=-1)
    att = jnp.dot(hcat, aw_ref[0], preferred_element_type=jnp.float32) \
        + ab_ref[0]
    out_ref[0] = hcat + jax.nn.sigmoid(att) * hcat


def _post_layer(o, gb, h, xw, xb, cw, cb, aw, ab):
    return pl.pallas_call(
        _post_kernel,
        grid=(P, N // BN),
        in_specs=[
            pl.BlockSpec((P, 1, BN, H), lambda d, j: (0, d, j, 0)),
            pl.BlockSpec((P, 1, 1, H), lambda d, j: (0, d, 0, 0)),
            pl.BlockSpec((1, BN, H), lambda d, j: (d, j, 0)),
            pl.BlockSpec((1, H, HC), lambda d, j: (d, 0, 0)),
            pl.BlockSpec((1, 1, HC), lambda d, j: (d, 0, 0)),
            pl.BlockSpec((1, H, HC), lambda d, j: (d, 0, 0)),
            pl.BlockSpec((1, 1, HC), lambda d, j: (d, 0, 0)),
            pl.BlockSpec((1, H, H), lambda d, j: (d, 0, 0)),
            pl.BlockSpec((1, 1, H), lambda d, j: (d, 0, 0)),
        ],
        out_specs=pl.BlockSpec((1, BN, H), lambda d, j: (d, j, 0)),
        out_shape=jax.ShapeDtypeStruct((P, N, H), jnp.float32),
    )(o, gb, h, xw, xb[:, None, :], cw, cb[:, None, :], aw, ab[:, None, :])


def _sc_edge_kernel(als, ald, msp, ea, srcg, dstl, xs, out_hbm,
                    als_v, ald_v, m_v, ea_v, srcg_v, dstl_v, e_v, den_v,
                    iota_v, zden_v, w_v, rows_v, zero_v,
                    den_sh, out_sh,
                    g0, g1, g2, g3, s0, s1, s2, s3):
    gsems = (g0, g1, g2, g3)
    ssems = (s0, s1, s2, s3)
    c = lax.axis_index("c")
    s = lax.axis_index("s")
    zero16 = jnp.zeros((16,), jnp.float32)
    iota16 = lax.iota(jnp.int32, 16)

    def _z(r, _):
        for f in range(4):
            zero_v[r, pl.ds(f * 16, 16)] = zero16
        return 0
    lax.fori_loop(0, 128, _z, 0)

    def _z2(r, _):
        zden_v[r, :] = zero16
        return 0
    lax.fori_loop(0, 40, _z2, 0)
    for z in range(5):
        for j in range(8):
            iota_v[z, pl.ds(j * 16, 16)] = iota16 + (z * 128 + j * 16)

    def per_et(i, _):
        et = 2 * i + c

        @pl.when(et < P * P)
        def _():
            hs = [
                pltpu.async_copy(als.at[et], als_v, g0),
                pltpu.async_copy(ald.at[et], ald_v, g0),
                pltpu.async_copy(msp.at[et], m_v, g0),
                pltpu.async_copy(ea.at[et, pl.ds(s * 2048, 2048)], ea_v, g0),
                pltpu.async_copy(srcg.at[et, pl.ds(s * NBK, NBK)],
                                 srcg_v, g1),
                pltpu.async_copy(dstl.at[et, pl.ds(s * NBK, NBK)],
                                 dstl_v, g1),
                pltpu.async_copy(zden_v, den_sh.at[pl.ds(s * 40, 40)], g2),
            ] + [
                pltpu.async_copy(
                    zero_v, out_sh.at[pl.ds(s * NSL + z * 128, 128)], g3)
                for z in range(NSL // 128)
            ]

            def _zd(r, _):
                den_v[r, :] = zero16
                return 0
            lax.fori_loop(0, NP // 16, _zd, 0)
            for hh in hs:
                hh.wait()
            plsc.subcore_barrier()

            m16 = m_v[...]
            etN = et * N

            def _p1(b, _):
                for k in range(8):
                    sl = pl.ds(k * 16, 16)
                    srcv = srcg_v[b, sl] - etN
                    dstv = dstl_v[b, sl]
                    ag = plsc.load_gather(als_v, [srcv])
                    dg = plsc.load_gather(ald_v, [dstv])
                    a = ag + dg + ea_v[pl.ds(b * 128 + k * 16, 16)]
                    a = jnp.where(a >= 0.0, a, 0.2 * a)
                    e = jnp.exp(a - m16)
                    pos = lax.iota(jnp.int32, 16) + (
                        s * 2048 + b * 128 + k * 16)
                    e = jnp.where(pos < EPT, e, 0.0)
                    e_v[pl.ds(b * 128 + k * 16, 16)] = e
                    plsc.addupdate_scatter(
                        den_v, [dstv >> 4, dstv & 15], e)
                return 0
            lax.fori_loop(0, NBK, _p1, 0)

            mh = [pltpu.async_copy(den_v.at[pl.ds(z * 128, 128)],
                                   den_sh.at[iota_v.at[z]], g0, add=True)
                  for z in range(5)]
            for hh in mh:
                hh.wait()
            plsc.subcore_barrier()
            pltpu.sync_copy(den_sh, den_v)

            def _w_and_scale(b, buf):
                for k in range(8):
                    sl = pl.ds(k * 16, 16)
                    dstv = dstl_v[b, sl]
                    dg = plsc.load_gather(den_v, [dstv >> 4, dstv & 15])
                    w_v[sl] = e_v[pl.ds(b * 128 + k * 16, 16)] / (
                        dg + 1e-16)

                def _scale(i2, _):
                    wsp = plsc.load_gather(
                        w_v, [jnp.full((16,), i2, jnp.int32)])
                    for f in range(4):
                        fsl = pl.ds(f * 16, 16)
                        buf[i2, fsl] = buf[i2, fsl] * wsp
                    return 0
                lax.fori_loop(0, 128, _scale, 0)

            gh = [None] * 4
            sh = [None] * 4
            for b in (0, 1):
                gh[b] = pltpu.async_copy(xs.at[srcg_v.at[b]], rows_v.at[b],
                                         gsems[b])
            for b in range(NBK):
                i = b % 4
                gh[i].wait()
                _w_and_scale(b, rows_v.at[i])
                sh[i] = pltpu.async_copy(rows_v.at[i],
                                         out_sh.at[dstl_v.at[b]],
                                         ssems[i], add=True)
                if b + 2 < NBK:
                    j = (b + 2) % 4
                    if sh[j] is not None:
                        sh[j].wait()
                    gh[j] = pltpu.async_copy(xs.at[srcg_v.at[b + 2]],
                                             rows_v.at[j], gsems[j])
            for i in range(4):
                sh[i].wait()
            plsc.subcore_barrier()
            pltpu.sync_copy(out_sh.at[pl.ds(s * NSL, NSL)],
                            out_hbm.at[et, pl.ds(s * NSL, NSL)])
            plsc.subcore_barrier()
        return 0

    lax.fori_loop(0, ETC, per_et, 0)


def _sc_edge_call(als, ald, msp, ea, srcg, dstl, xs_flat):
    f32 = jnp.float32
    mesh = plsc.VectorSubcoreMesh(core_axis_name="c", subcore_axis_name="s")
    return pl.kernel(
        _sc_edge_kernel,
        out_type=jax.ShapeDtypeStruct((P * P, NP, H), f32),
        mesh=mesh,
        compiler_params=pltpu.CompilerParams(needs_layout_passes=False,
                                             use_tc_tiling_on_sc=False),
        scratch_types=[
            pltpu.VMEM((NP,), f32),
            pltpu.VMEM((NP,), f32),
            pltpu.VMEM((16,), f32),
            pltpu.VMEM((2048,), f32),
            pltpu.VMEM((NBK, 128), jnp.int32),
            pltpu.VMEM((NBK, 128), jnp.int32),
            pltpu.VMEM((2048,), f32),
            pltpu.VMEM((NP // 16, 16), f32),
            pltpu.VMEM((5, 128), jnp.int32),
            pltpu.VMEM((40, 16), f32),
            pltpu.VMEM((128,), f32),
            pltpu.VMEM((4, 128, H), f32),
            pltpu.VMEM((128, H), f32),
            pltpu.VMEM_SHARED((NP // 16, 16), f32),
            pltpu.VMEM_SHARED((NP, H), f32),
        ] + [pltpu.SemaphoreType.DMA] * 8,
    )(als, ald, msp, ea, srcg, dstl, xs_flat)


def _edge_phase(h, l, ea_pad, srcg, dstl, gat_W, gat_as, gat_ad):
    xs = _xs_all(h, gat_W[l])
    vd = jnp.einsum('ehk,ek->eh', gat_W[l], gat_ad[l])
    al_s = jnp.einsum('enh,eh->en', xs, gat_as[l])
    al_d = jnp.einsum('enh,eh->en', h[jnp.arange(P * P) % P], vd)
    m = _leaky(al_s.max(1) + al_d.max(1) + ea_pad[l].max(1), 0.2)
    msp = jnp.broadcast_to(m[:, None], (P * P, 16))
    als_p = jnp.pad(al_s, ((0, 0), (0, NP - N)))
    ald_p = jnp.pad(al_d, ((0, 0), (0, NP - N)))
    out = _sc_edge_call(als_p, ald_p, msp, ea_pad[l], srcg, dstl,
                        xs.reshape(P * P * N, H))
    return out[:, :N, :].reshape(P, P, N, H)


def kernel(x, cond, edge_index, edge_attr, emb_W, emb_b, sl_W, sl_b, fin_W,
           fin_b, gat_W, gat_as, gat_ad, gat_We, gat_ae, gat_b,
           xc_W, xc_b, cc_W, cc_b, at_W, at_b):
    h = _mm3(jnp.concatenate([x, cond], axis=-1), emb_W, emb_b)
    ve = jnp.einsum('ledh,leh->led', gat_We, gat_ae)
    ea_all = jnp.einsum('ked,lkd->lke', edge_attr, ve)
    ea_pad = jnp.pad(ea_all, ((0, 0), (0, 0), (0, EPP - EPT)))
    et_off = (jnp.arange(P * P, dtype=jnp.int32) * N)[:, None]
    srcg = (jnp.pad(edge_index[:, 0], ((0, 0), (0, EPP - EPT)))
            + et_off).reshape(P * P, EPP // 128, 128)
    dstl = jnp.pad(edge_index[:, 1],
                   ((0, 0), (0, EPP - EPT))).reshape(P * P, EPP // 128, 128)
    for l in range(L):
        h = _mm3(h, sl_W[l], sl_b[l], act=lambda v: _leaky(v, 0.01))
        o = _edge_phase(h, l, ea_pad, srcg, dstl, gat_W, gat_as, gat_ad)
        h = _post_layer(o, gat_b[l].reshape(P, P, 1, H), h,
                        xc_W[l], xc_b[l], cc_W[l], cc_b[l], at_W[l], at_b[l])
    out = _mm3(h, fin_W, fin_b)
    return out.reshape(P, N, Z, 2)


# speedup vs baseline: 21.4325x; 1.0049x over previous
import functools

import jax
import jax.numpy as jnp
from jax import lax
from jax.experimental import pallas as pl
from jax.experimental.pallas import tpu as pltpu
from jax.experimental.pallas import tpu_sc as plsc

P = 5
N = 10000
EPT = 32000
T = 6
C = 10
ED = 64
H = 64
HC = 32
DE = 4
Z = 16
L = 3
BN = 2000

NP = 10240
EPP = 32768
NSL = 640
NBK = 16
ETC = 13


def _leaky(x, s):
    return jnp.where(x >= 0, x, s * x)


def _mm_kernel(act):
    def k(h_ref, w_ref, b_ref, o_ref):
        o_ref[0] = act(jnp.dot(h_ref[0], w_ref[0],
                               preferred_element_type=jnp.float32) + b_ref[0])
    return k


def _mm3(h, W, b, act=lambda x: x):
    B, n, D = h.shape
    E = W.shape[-1]
    return pl.pallas_call(
        _mm_kernel(act),
        grid=(B, n // BN),
        in_specs=[
            pl.BlockSpec((1, BN, D), lambda i, j: (i, j, 0)),
            pl.BlockSpec((1, D, E), lambda i, j: (i, 0, 0)),
            pl.BlockSpec((1, 1, E), lambda i, j: (i, 0, 0)),
        ],
        out_specs=pl.BlockSpec((1, BN, E), lambda i, j: (i, j, 0)),
        out_shape=jax.ShapeDtypeStruct((B, n, E), jnp.float32),
    )(h, W, b[:, None, :])


def _xs_kernel(h_ref, w_ref, o_ref):
    o_ref[0] = jnp.dot(h_ref[0], w_ref[0], preferred_element_type=jnp.float32)


def _xs_all(h, gWl):
    return pl.pallas_call(
        _xs_kernel,
        grid=(P * P, N // BN),
        in_specs=[
            pl.BlockSpec((1, BN, H), lambda e, j: (e // P, j, 0)),
            pl.BlockSpec((1, H, H), lambda e, j: (e, 0, 0)),
        ],
        out_specs=pl.BlockSpec((1, BN, H), lambda e, j: (e, j, 0)),
        out_shape=jax.ShapeDtypeStruct((P * P, N, H), jnp.float32),
    )(h, gWl)


def _post_kernel(o_ref, b_ref, h_ref, xw_ref, xb_ref, cw_ref, cb_ref,
                 aw_ref, ab_ref, out_ref):
    m = jnp.max(o_ref[:, 0] + b_ref[:, 0], axis=0)
    comm = _leaky(m, 0.01)
    a = jnp.dot(h_ref[0], xw_ref[0], preferred_element_type=jnp.float32) \
        + xb_ref[0]
    b = jnp.dot(comm, cw_ref[0], preferred_element_type=jnp.float32) \
        + cb_ref[0]
    hcat = jnp.concatenate([a, b], axis=-1)
    att = jnp.dot(hcat, aw_ref[0], preferred_element_type=jnp.float32) \
        + ab_ref[0]
    out_ref[0] = hcat + jax.nn.sigmoid(att) * hcat


def _post_layer(o, gb, h, xw, xb, cw, cb, aw, ab):
    return pl.pallas_call(
        _post_kernel,
        grid=(P, N // BN),
        in_specs=[
            pl.BlockSpec((P, 1, BN, H), lambda d, j: (0, d, j, 0)),
            pl.BlockSpec((P, 1, 1, H), lambda d, j: (0, d, 0, 0)),
            pl.BlockSpec((1, BN, H), lambda d, j: (d, j, 0)),
            pl.BlockSpec((1, H, HC), lambda d, j: (d, 0, 0)),
            pl.BlockSpec((1, 1, HC), lambda d, j: (d, 0, 0)),
            pl.BlockSpec((1, H, HC), lambda d, j: (d, 0, 0)),
            pl.BlockSpec((1, 1, HC), lambda d, j: (d, 0, 0)),
            pl.BlockSpec((1, H, H), lambda d, j: (d, 0, 0)),
            pl.BlockSpec((1, 1, H), lambda d, j: (d, 0, 0)),
        ],
        out_specs=pl.BlockSpec((1, BN, H), lambda d, j: (d, j, 0)),
        out_shape=jax.ShapeDtypeStruct((P, N, H), jnp.float32),
    )(o, gb, h, xw, xb[:, None, :], cw, cb[:, None, :], aw, ab[:, None, :])


def _sc_edge_kernel(als, ald, msp, ea, srcg, dstl, xs, out_hbm,
                    als_v, ald_v, m_v, ea_v, srcg_v, dstl_v, e_v, den_v,
                    iota_v, zden_v, w_v, rows_v, zero_v,
                    den_sh, out_sh,
                    g0, g1, g2, g3, s0, s1, s2, s3):
    gsems = (g0, g1, g2, g3)
    ssems = (s0, s1, s2, s3)
    c = lax.axis_index("c")
    s = lax.axis_index("s")
    zero16 = jnp.zeros((16,), jnp.float32)
    iota16 = lax.iota(jnp.int32, 16)

    def _z(r, _):
        for f in range(4):
            zero_v[r, pl.ds(f * 16, 16)] = zero16
        return 0
    lax.fori_loop(0, 128, _z, 0)

    def _z2(r, _):
        zden_v[r, :] = zero16
        return 0
    lax.fori_loop(0, 40, _z2, 0)
    for z in range(5):
        for j in range(8):
            iota_v[z, pl.ds(j * 16, 16)] = iota16 + (z * 128 + j * 16)

    def per_et(i, _):
        et = 2 * i + c

        @pl.when(et < P * P)
        def _():
            hs = [
                pltpu.async_copy(als.at[et], als_v, g0),
                pltpu.async_copy(ald.at[et], ald_v, g0),
                pltpu.async_copy(msp.at[et], m_v, g0),
                pltpu.async_copy(ea.at[et, pl.ds(s * 2048, 2048)], ea_v, g0),
                pltpu.async_copy(srcg.at[et, pl.ds(s * NBK, NBK)],
                                 srcg_v, g1),
                pltpu.async_copy(dstl.at[et, pl.ds(s * NBK, NBK)],
                                 dstl_v, g1),
                pltpu.async_copy(zden_v, den_sh.at[pl.ds(s * 40, 40)], g2),
            ] + [
                pltpu.async_copy(
                    zero_v, out_sh.at[pl.ds(s * NSL + z * 128, 128)], g3)
                for z in range(NSL // 128)
            ]

            def _zd(r, _):
                for u in range(8):
                    den_v[r * 8 + u, :] = zero16
                return 0
            lax.fori_loop(0, NP // 128, _zd, 0)
            for hh in hs:
                hh.wait()
            plsc.subcore_barrier()

            m16 = m_v[...]
            etN = et * N

            def _p1(b, _):
                for k in range(8):
                    sl = pl.ds(k * 16, 16)
                    srcv = srcg_v[b, sl] - etN
                    dstv = dstl_v[b, sl]
                    ag = plsc.load_gather(als_v, [srcv])
                    dg = plsc.load_gather(ald_v, [dstv])
                    a = ag + dg + ea_v[pl.ds(b * 128 + k * 16, 16)]
                    a = jnp.where(a >= 0.0, a, 0.2 * a)
                    e = jnp.exp(a - m16)
                    pos = lax.iota(jnp.int32, 16) + (
                        s * 2048 + b * 128 + k * 16)
                    e = jnp.where(pos < EPT, e, 0.0)
                    e_v[pl.ds(b * 128 + k * 16, 16)] = e
                    plsc.addupdate_scatter(
                        den_v, [dstv >> 4, dstv & 15], e)
                return 0
            lax.fori_loop(0, NBK, _p1, 0)

            mh = [pltpu.async_copy(den_v.at[pl.ds(z * 128, 128)],
                                   den_sh.at[iota_v.at[z]], g0, add=True)
                  for z in range(5)]
            for hh in mh:
                hh.wait()
            plsc.subcore_barrier()
            pltpu.sync_copy(den_sh, den_v)

            def _w_and_scale(b, buf):
                for k in range(8):
                    sl = pl.ds(k * 16, 16)
                    dstv = dstl_v[b, sl]
                    dg = plsc.load_gather(den_v, [dstv >> 4, dstv & 15])
                    w_v[sl] = e_v[pl.ds(b * 128 + k * 16, 16)] / (
                        dg + 1e-16)

                def _scale(j, _):
                    for u in range(8):
                        i2 = j * 8 + u
                        wsp = plsc.load_gather(
                            w_v, [jnp.full((16,), i2, jnp.int32)])
                        for f in range(4):
                            fsl = pl.ds(f * 16, 16)
                            buf[i2, fsl] = buf[i2, fsl] * wsp
                    return 0
                lax.fori_loop(0, 16, _scale, 0)

            gh = [None] * 4
            sh = [None] * 4
            for b in (0, 1):
                gh[b] = pltpu.async_copy(xs.at[srcg_v.at[b]], rows_v.at[b],
                                         gsems[b])
            for b in range(NBK):
                i = b % 4
                gh[i].wait()
                _w_and_scale(b, rows_v.at[i])
                sh[i] = pltpu.async_copy(rows_v.at[i],
                                         out_sh.at[dstl_v.at[b]],
                                         ssems[i], add=True)
                if b + 2 < NBK:
                    j = (b + 2) % 4
                    if sh[j] is not None:
                        sh[j].wait()
                    gh[j] = pltpu.async_copy(xs.at[srcg_v.at[b + 2]],
                                             rows_v.at[j], gsems[j])
            for i in range(4):
                sh[i].wait()
            plsc.subcore_barrier()
            pltpu.sync_copy(out_sh.at[pl.ds(s * NSL, NSL)],
                            out_hbm.at[et, pl.ds(s * NSL, NSL)])
            plsc.subcore_barrier()
        return 0

    lax.fori_loop(0, ETC, per_et, 0)


def _sc_edge_call(als, ald, msp, ea, srcg, dstl, xs_flat):
    f32 = jnp.float32
    mesh = plsc.VectorSubcoreMesh(core_axis_name="c", subcore_axis_name="s")
    return pl.kernel(
        _sc_edge_kernel,
        out_type=jax.ShapeDtypeStruct((P * P, NP, H), f32),
        mesh=mesh,
        compiler_params=pltpu.CompilerParams(needs_layout_passes=False,
                                             use_tc_tiling_on_sc=False),
        scratch_types=[
            pltpu.VMEM((NP,), f32),
            pltpu.VMEM((NP,), f32),
            pltpu.VMEM((16,), f32),
            pltpu.VMEM((2048,), f32),
            pltpu.VMEM((NBK, 128), jnp.int32),
            pltpu.VMEM((NBK, 128), jnp.int32),
            pltpu.VMEM((2048,), f32),
            pltpu.VMEM((NP // 16, 16), f32),
            pltpu.VMEM((5, 128), jnp.int32),
            pltpu.VMEM((40, 16), f32),
            pltpu.VMEM((128,), f32),
            pltpu.VMEM((4, 128, H), f32),
            pltpu.VMEM((128, H), f32),
            pltpu.VMEM_SHARED((NP // 16, 16), f32),
            pltpu.VMEM_SHARED((NP, H), f32),
        ] + [pltpu.SemaphoreType.DMA] * 8,
    )(als, ald, msp, ea, srcg, dstl, xs_flat)


def _edge_phase(h, l, ea_pad, srcg, dstl, gat_W, gat_as, gat_ad):
    xs = _xs_all(h, gat_W[l])
    vd = jnp.einsum('ehk,ek->eh', gat_W[l], gat_ad[l])
    al_s = jnp.einsum('enh,eh->en', xs, gat_as[l])
    al_d = jnp.einsum('enh,eh->en', h[jnp.arange(P * P) % P], vd)
    m = _leaky(al_s.max(1) + al_d.max(1) + ea_pad[l].max(1), 0.2)
    msp = jnp.broadcast_to(m[:, None], (P * P, 16))
    als_p = jnp.pad(al_s, ((0, 0), (0, NP - N)))
    ald_p = jnp.pad(al_d, ((0, 0), (0, NP - N)))
    out = _sc_edge_call(als_p, ald_p, msp, ea_pad[l], srcg, dstl,
                        xs.reshape(P * P * N, H))
    return out[:, :N, :].reshape(P, P, N, H)


def kernel(x, cond, edge_index, edge_attr, emb_W, emb_b, sl_W, sl_b, fin_W,
           fin_b, gat_W, gat_as, gat_ad, gat_We, gat_ae, gat_b,
           xc_W, xc_b, cc_W, cc_b, at_W, at_b):
    h = _mm3(jnp.concatenate([x, cond], axis=-1), emb_W, emb_b)
    ve = jnp.einsum('ledh,leh->led', gat_We, gat_ae)
    ea_all = jnp.einsum('ked,lkd->lke', edge_attr, ve)
    ea_pad = jnp.pad(ea_all, ((0, 0), (0, 0), (0, EPP - EPT)))
    et_off = (jnp.arange(P * P, dtype=jnp.int32) * N)[:, None]
    srcg = (jnp.pad(edge_index[:, 0], ((0, 0), (0, EPP - EPT)))
            + et_off).reshape(P * P, EPP // 128, 128)
    dstl = jnp.pad(edge_index[:, 1],
                   ((0, 0), (0, EPP - EPT))).reshape(P * P, EPP // 128, 128)
    for l in range(L):
        h = _mm3(h, sl_W[l], sl_b[l], act=lambda v: _leaky(v, 0.01))
        o = _edge_phase(h, l, ea_pad, srcg, dstl, gat_W, gat_as, gat_ad)
        h = _post_layer(o, gat_b[l].reshape(P, P, 1, H), h,
                        xc_W[l], xc_b[l], cc_W[l], cc_b[l], at_W[l], at_b[l])
    out = _mm3(h, fin_W, fin_b)
    return out.reshape(P, N, Z, 2)


# speedup vs baseline: 23.7406x; 1.1077x over previous
import functools

import jax
import jax.numpy as jnp
from jax import lax
from jax.experimental import pallas as pl
from jax.experimental.pallas import tpu as pltpu
from jax.experimental.pallas import tpu_sc as plsc

P = 5
N = 10000
EPT = 32000
T = 6
C = 10
ED = 64
H = 64
HC = 32
DE = 4
Z = 16
L = 3
BN = 2000

NP = 10240
EPP = 32768
NSL = 640
NBK = 16
ETC = 13


def _leaky(x, s):
    return jnp.where(x >= 0, x, s * x)


def _mm_kernel(act):
    def k(h_ref, w_ref, b_ref, o_ref):
        o_ref[0] = act(jnp.dot(h_ref[0], w_ref[0],
                               preferred_element_type=jnp.float32) + b_ref[0])
    return k


def _mm3(h, W, b, act=lambda x: x):
    B, n, D = h.shape
    E = W.shape[-1]
    return pl.pallas_call(
        _mm_kernel(act),
        grid=(B, n // BN),
        in_specs=[
            pl.BlockSpec((1, BN, D), lambda i, j: (i, j, 0)),
            pl.BlockSpec((1, D, E), lambda i, j: (i, 0, 0)),
            pl.BlockSpec((1, 1, E), lambda i, j: (i, 0, 0)),
        ],
        out_specs=pl.BlockSpec((1, BN, E), lambda i, j: (i, j, 0)),
        out_shape=jax.ShapeDtypeStruct((B, n, E), jnp.float32),
    )(h, W, b[:, None, :])


def _xs_kernel(h_ref, w_ref, o_ref):
    o_ref[0] = jnp.dot(h_ref[0], w_ref[0], preferred_element_type=jnp.float32)


def _xs_all(h, gWl):
    return pl.pallas_call(
        _xs_kernel,
        grid=(P * P, N // BN),
        in_specs=[
            pl.BlockSpec((1, BN, H), lambda e, j: (e // P, j, 0)),
            pl.BlockSpec((1, H, H), lambda e, j: (e, 0, 0)),
        ],
        out_specs=pl.BlockSpec((1, BN, H), lambda e, j: (e, j, 0)),
        out_shape=jax.ShapeDtypeStruct((P * P, N, H), jnp.float32),
    )(h, gWl)


def _post_kernel(o_ref, b_ref, h_ref, xw_ref, xb_ref, cw_ref, cb_ref,
                 aw_ref, ab_ref, out_ref):
    m = jnp.max(o_ref[:, 0] + b_ref[:, 0], axis=0)
    comm = _leaky(m, 0.01)
    a = jnp.dot(h_ref[0], xw_ref[0], preferred_element_type=jnp.float32) \
        + xb_ref[0]
    b = jnp.dot(comm, cw_ref[0], preferred_element_type=jnp.float32) \
        + cb_ref[0]
    hcat = jnp.concatenate([a, b], axis=-1)
    att = jnp.dot(hcat, aw_ref[0], preferred_element_type=jnp.float32) \
        + ab_ref[0]
    out_ref[0] = hcat + jax.nn.sigmoid(att) * hcat


def _post_layer(o, gb, h, xw, xb, cw, cb, aw, ab):
    return pl.pallas_call(
        _post_kernel,
        grid=(P, N // BN),
        in_specs=[
            pl.BlockSpec((P, 1, BN, H), lambda d, j: (0, d, j, 0)),

            pl.BlockSpec((P, 1, 1, H), lambda d, j: (0, d, 0, 0)),
            pl.BlockSpec((1, BN, H), lambda d, j: (d, j, 0)),
            pl.BlockSpec((1, H, HC), lambda d, j: (d, 0, 0)),
            pl.BlockSpec((1, 1, HC), lambda d, j: (d, 0, 0)),
            pl.BlockSpec((1, H, HC), lambda d, j: (d, 0, 0)),
            pl.BlockSpec((1, 1, HC), lambda d, j: (d, 0, 0)),
            pl.BlockSpec((1, H, H), lambda d, j: (d, 0, 0)),
            pl.BlockSpec((1, 1, H), lambda d, j: (d, 0, 0)),
        ],
        out_specs=pl.BlockSpec((1, BN, H), lambda d, j: (d, j, 0)),
        out_shape=jax.ShapeDtypeStruct((P, N, H), jnp.float32),
    )(o, gb, h, xw, xb[:, None, :], cw, cb[:, None, :], aw, ab[:, None, :])


def _sc_edge_kernel(al2, msp, ea, srcg, dstl, xs, out_hbm,
                    als_v, ald_v, m_v, ea_v, srcg_v, dstl_v, e_v, den_v,
                    iota_v, zden_v, w_v, rows_v, zero_v,
                    den_sh, out_sh,
                    g0, g1, g2, g3, s0, s1, s2, s3):
    gsems = (g0, g1, g2, g3)
    ssems = (s0, s1, s2, s3)
    c = lax.axis_index("c")
    s = lax.axis_index("s")
    zero16 = jnp.zeros((16,), jnp.float32)
    iota16 = lax.iota(jnp.int32, 16)

    def _z(r, _):
        for f in range(4):
            zero_v[r, pl.ds(f * 16, 16)] = zero16
        return 0
    lax.fori_loop(0, 128, _z, 0)

    def _z2(r, _):
        zden_v[r, :] = zero16
        return 0
    lax.fori_loop(0, 40, _z2, 0)
    for z in range(5):
        for j in range(8):
            iota_v[z, pl.ds(j * 16, 16)] = iota16 + (z * 128 + j * 16)

    def per_et(i, _):
        et = 2 * i + c

        @pl.when(et < P * P)
        def _():
            hs = [
                pltpu.async_copy(al2.at[et // P, et % P],
                                 als_v.at[pl.ds(0, N)], g0),
                pltpu.async_copy(al2.at[et % P, P + et // P],
                                 ald_v.at[pl.ds(0, N)], g0),
                pltpu.async_copy(msp.at[et], m_v, g0),
                pltpu.async_copy(ea.at[et, pl.ds(s * 2048, 2048)], ea_v, g0),
                pltpu.async_copy(srcg.at[et, pl.ds(s * NBK, NBK)],
                                 srcg_v, g1),
                pltpu.async_copy(dstl.at[et, pl.ds(s * NBK, NBK)],
                                 dstl_v, g1),
                pltpu.async_copy(zden_v, den_sh.at[pl.ds(s * 40, 40)], g2),
            ] + [
                pltpu.async_copy(
                    zero_v, out_sh.at[pl.ds(s * NSL + z * 128, 128)], g3)
                for z in range(NSL // 128)
            ]

            def _zd(r, _):
                for u in range(8):
                    den_v[r * 8 + u, :] = zero16
                return 0
            lax.fori_loop(0, NP // 128, _zd, 0)
            for hh in hs:
                hh.wait()
            plsc.subcore_barrier()

            m16 = m_v[...]
            etN = et * N

            def _p1(b, _):
                for k in range(8):
                    sl = pl.ds(k * 16, 16)
                    srcv = srcg_v[b, sl] - etN
                    dstv = dstl_v[b, sl]
                    ag = plsc.load_gather(als_v, [srcv])
                    dg = plsc.load_gather(ald_v, [dstv])
                    a = ag + dg + ea_v[pl.ds(b * 128 + k * 16, 16)]
                    a = jnp.where(a >= 0.0, a, 0.2 * a)
                    e = jnp.exp(a - m16)
                    pos = lax.iota(jnp.int32, 16) + (
                        s * 2048 + b * 128 + k * 16)
                    e = jnp.where(pos < EPT, e, 0.0)
                    e_v[pl.ds(b * 128 + k * 16, 16)] = e
                    plsc.addupdate_scatter(
                        den_v, [dstv >> 4, dstv & 15], e)
                return 0
            lax.fori_loop(0, NBK, _p1, 0)

            mh = [pltpu.async_copy(den_v.at[pl.ds(z * 128, 128)],
                                   den_sh.at[iota_v.at[z]], g0, add=True)
                  for z in range(5)]
            for hh in mh:
                hh.wait()
            plsc.subcore_barrier()
            pltpu.sync_copy(den_sh, den_v)

            def _w_and_scale(b, buf):
                for k in range(8):
                    sl = pl.ds(k * 16, 16)
                    dstv = dstl_v[b, sl]
                    dg = plsc.load_gather(den_v, [dstv >> 4, dstv & 15])
                    w_v[sl] = e_v[pl.ds(b * 128 + k * 16, 16)] / (
                        dg + 1e-16)

                def _scale(j, _):
                    for u in range(8):
                        i2 = j * 8 + u
                        wsp = plsc.load_gather(
                            w_v, [jnp.full((16,), i2, jnp.int32)])
                        for f in range(4):
                            fsl = pl.ds(f * 16, 16)
                            buf[i2, fsl] = buf[i2, fsl] * wsp
                    return 0
                lax.fori_loop(0, 16, _scale, 0)

            gh = [None] * 4
            sh = [None] * 4
            for b in (0, 1):
                gh[b] = pltpu.async_copy(xs.at[srcg_v.at[b]], rows_v.at[b],
                                         gsems[b])
            for b in range(NBK):
                i = b % 4
                gh[i].wait()
                _w_and_scale(b, rows_v.at[i])
                sh[i] = pltpu.async_copy(rows_v.at[i],
                                         out_sh.at[dstl_v.at[b]],
                                         ssems[i], add=True)
                if b + 2 < NBK:
                    j = (b + 2) % 4
                    if sh[j] is not None:
                        sh[j].wait()
                    gh[j] = pltpu.async_copy(xs.at[srcg_v.at[b + 2]],
                                             rows_v.at[j], gsems[j])
            for i in range(4):
                sh[i].wait()
            plsc.subcore_barrier()
            pltpu.sync_copy(out_sh.at[pl.ds(s * NSL, NSL)],
                            out_hbm.at[et, pl.ds(s * NSL, NSL)])
            plsc.subcore_barrier()
        return 0

    lax.fori_loop(0, ETC, per_et, 0)


def _sc_edge_call(al2, msp, ea, srcg, dstl, xs_flat):
    f32 = jnp.float32
    mesh = plsc.VectorSubcoreMesh(core_axis_name="c", subcore_axis_name="s")
    return pl.kernel(
        _sc_edge_kernel,
        out_type=jax.ShapeDtypeStruct((P * P, NP, H), f32),
        mesh=mesh,
        compiler_params=pltpu.CompilerParams(needs_layout_passes=False,
                                             use_tc_tiling_on_sc=False),
        scratch_types=[
            pltpu.VMEM((NP,), f32),
            pltpu.VMEM((NP,), f32),
            pltpu.VMEM((16,), f32),
            pltpu.VMEM((2048,), f32),
            pltpu.VMEM((NBK, 128), jnp.int32),
            pltpu.VMEM((NBK, 128), jnp.int32),
            pltpu.VMEM((2048,), f32),
            pltpu.VMEM((NP // 16, 16), f32),
            pltpu.VMEM((5, 128), jnp.int32),
            pltpu.VMEM((40, 16), f32),
            pltpu.VMEM((128,), f32),
            pltpu.VMEM((4, 128, H), f32),
            pltpu.VMEM((128, H), f32),
            pltpu.VMEM_SHARED((NP // 16, 16), f32),
            pltpu.VMEM_SHARED((NP, H), f32),
        ] + [pltpu.SemaphoreType.DMA] * 8,
    )(al2, msp, ea, srcg, dstl, xs_flat)


def _edge_phase(h, l, ea_pad, srcg, dstl, gat_W, gat_as, gat_ad):
    xs = _xs_all(h, gat_W[l])
    va = jnp.einsum('ehk,ek->eh', gat_W[l], gat_as[l]).reshape(P, P, H)
    vd = jnp.einsum('ehk,ek->eh', gat_W[l], gat_ad[l]).reshape(P, P, H)
    VAB = jnp.pad(jnp.concatenate([va.transpose(0, 2, 1),
                                   vd.transpose(1, 2, 0)], axis=2),
                  ((0, 0), (0, 0), (0, 6)))
    AB = _mm3(h, VAB, jnp.zeros((P, 16), jnp.float32))
    al2 = AB.transpose(0, 2, 1)
    maxs = AB.max(1)
    s_i = jnp.arange(P * P) // P
    d_i = jnp.arange(P * P) % P
    m = _leaky(maxs[s_i, d_i] + maxs[d_i, 5 + s_i] + ea_pad[l].max(1), 0.2)
    msp = jnp.broadcast_to(m[:, None], (P * P, 16))
    out = _sc_edge_call(al2, msp, ea_pad[l], srcg, dstl,
                        xs.reshape(P * P * N, H))
    return out.reshape(P, P, NP, H)


def kernel(x, cond, edge_index, edge_attr, emb_W, emb_b, sl_W, sl_b, fin_W,
           fin_b, gat_W, gat_as, gat_ad, gat_We, gat_ae, gat_b,
           xc_W, xc_b, cc_W, cc_b, at_W, at_b):
    h = _mm3(jnp.concatenate([x, cond], axis=-1), emb_W, emb_b)
    ve = jnp.einsum('ledh,leh->led', gat_We, gat_ae)
    ea_all = jnp.einsum('ked,lkd->lke', edge_attr, ve)
    ea_pad = jnp.pad(ea_all, ((0, 0), (0, 0), (0, EPP - EPT)))
    et_off = (jnp.arange(P * P, dtype=jnp.int32) * N)[:, None]
    srcg = (jnp.pad(edge_index[:, 0], ((0, 0), (0, EPP - EPT)))
            + et_off).reshape(P * P, EPP // 128, 128)
    dstl = jnp.pad(edge_index[:, 1],
                   ((0, 0), (0, EPP - EPT))).reshape(P * P, EPP // 128, 128)
    for l in range(L):
        h = _mm3(h, sl_W[l], sl_b[l], act=lambda v: _leaky(v, 0.01))
        o = _edge_phase(h, l, ea_pad, srcg, dstl, gat_W, gat_as, gat_ad)
        h = _post_layer(o, gat_b[l].reshape(P, P, 1, H), h,
                        xc_W[l], xc_b[l], cc_W[l], cc_b[l], at_W[l], at_b[l])
    out = _mm3(h, fin_W, fin_b)
    return out.reshape(P, N, Z, 2)
